# baseline (device time: 20329 ns/iter reference)
import jax
import jax.numpy as jnp
from jax import lax
from jax.experimental import pallas as pl
from jax.experimental.pallas import tpu as pltpu

N_DEV = 4
EPS = 1e-5


def kernel(x, dy, gamma):
    del gamma
    m, d = x.shape
    half = m // 2

    def body(x_ref, dy_ref, out_ref, comm_ref, send_sems, recv_sems):
        my_x = lax.axis_index("x")
        my_y = lax.axis_index("y")
        my_pos = my_x * 2 + my_y
        peers = [(1 - my_x, my_y), (my_x, 1 - my_y), (1 - my_x, 1 - my_y)]

        barrier = pltpu.get_barrier_semaphore()
        for (px, py) in peers:
            pl.semaphore_signal(
                barrier, inc=1,
                device_id=(px, py), device_id_type=pl.DeviceIdType.MESH,
            )
        pl.semaphore_wait(barrier, 3)

        row0 = my_y * half
        xb = x_ref[pl.ds(row0, half), :]
        dyb = dy_ref[pl.ds(row0, half), :]
        mu = jnp.mean(xb, axis=-1, keepdims=True)
        var = jnp.mean(jnp.square(xb - mu), axis=-1, keepdims=True)
        xhat = (xb - mu) * lax.rsqrt(var + EPS)
        dgamma = jnp.sum(dyb * xhat, axis=0, keepdims=True)
        dbeta = jnp.sum(dyb, axis=0, keepdims=True)
        comm_ref[my_pos] = jnp.concatenate([dgamma, dbeta], axis=0)

        sends = []
        for (px, py) in peers:
            q = px * 2 + py
            rdma = pltpu.make_async_remote_copy(
                src_ref=comm_ref.at[my_pos],
                dst_ref=comm_ref.at[my_pos],
                send_sem=send_sems.at[q],
                recv_sem=recv_sems.at[my_pos],
                device_id=(px, py),
                device_id_type=pl.DeviceIdType.MESH,
            )
            rdma.start()
            sends.append(rdma)

        for (px, py) in peers:
            q = px * 2 + py
            recv = pltpu.make_async_remote_copy(
                src_ref=comm_ref.at[q],
                dst_ref=comm_ref.at[q],
                send_sem=send_sems.at[q],
                recv_sem=recv_sems.at[q],
                device_id=(px, py),
                device_id_type=pl.DeviceIdType.MESH,
            )
            recv.wait_recv()

        for rdma in sends:
            rdma.wait_send()

        out_ref[:, :] = (
            comm_ref[0] + comm_ref[1] + comm_ref[2] + comm_ref[3]
        )

    return pl.pallas_call(
        body,
        out_shape=jax.ShapeDtypeStruct((2, d), jnp.float32),
        in_specs=[
            pl.BlockSpec(memory_space=pltpu.VMEM),
            pl.BlockSpec(memory_space=pltpu.VMEM),
        ],
        out_specs=pl.BlockSpec(memory_space=pltpu.VMEM),
        scratch_shapes=[
            pltpu.VMEM((N_DEV, 2, d), jnp.float32),
            pltpu.SemaphoreType.DMA((N_DEV,)),
            pltpu.SemaphoreType.DMA((N_DEV,)),
        ],
        compiler_params=pltpu.CompilerParams(collective_id=0),
    )(x, dy)


# device time: 18756 ns/iter; 1.0839x vs baseline; 1.0839x over previous
import jax
import jax.numpy as jnp
from jax import lax
from jax.experimental import pallas as pl
from jax.experimental.pallas import tpu as pltpu

N_DEV = 4
EPS = 1e-5
NCHUNK = 8


def kernel(x, dy, gamma):
    del gamma
    m, d = x.shape
    half = m // 2
    chunk = half // NCHUNK

    def body(x_ref, dy_ref, out_ref, acc_ref, comm_ref, send_sems, recv_sems):
        i = pl.program_id(0)
        my_x = lax.axis_index("x")
        my_y = lax.axis_index("y")
        my_pos = my_x * 2 + my_y
        peers = [(1 - my_x, my_y), (my_x, 1 - my_y), (1 - my_x, 1 - my_y)]

        barrier = pltpu.get_barrier_semaphore()

        @pl.when(i == 0)
        def _():
            for (px, py) in peers:
                pl.semaphore_signal(
                    barrier, inc=1,
                    device_id=(px, py), device_id_type=pl.DeviceIdType.MESH,
                )
            acc_ref[:, :] = jnp.zeros((2, d), jnp.float32)

        xb = x_ref[:, :]
        dyb = dy_ref[:, :]
        mu = jnp.mean(xb, axis=-1, keepdims=True)
        var = jnp.mean(jnp.square(xb - mu), axis=-1, keepdims=True)
        xhat = (xb - mu) * lax.rsqrt(var + EPS)
        dgamma = jnp.sum(dyb * xhat, axis=0, keepdims=True)
        dbeta = jnp.sum(dyb, axis=0, keepdims=True)
        acc_ref[:, :] += jnp.concatenate([dgamma, dbeta], axis=0)

        @pl.when(i == NCHUNK - 1)
        def _():
            pl.semaphore_wait(barrier, 3)
            comm_ref[my_pos] = acc_ref[:, :]

            sends = []
            for (px, py) in peers:
                q = px * 2 + py
                rdma = pltpu.make_async_remote_copy(
                    src_ref=comm_ref.at[my_pos],
                    dst_ref=comm_ref.at[my_pos],
                    send_sem=send_sems.at[q],
                    recv_sem=recv_sems.at[my_pos],
                    device_id=(px, py),
                    device_id_type=pl.DeviceIdType.MESH,
                )
                rdma.start()
                sends.append(rdma)

            for (px, py) in peers:
                q = px * 2 + py
                recv = pltpu.make_async_remote_copy(
                    src_ref=comm_ref.at[q],
                    dst_ref=comm_ref.at[q],
                    send_sem=send_sems.at[q],
                    recv_sem=recv_sems.at[q],
                    device_id=(px, py),
                    device_id_type=pl.DeviceIdType.MESH,
                )
                recv.wait_recv()

            for rdma in sends:
                rdma.wait_send()

            out_ref[:, :] = (
                comm_ref[0] + comm_ref[1] + comm_ref[2] + comm_ref[3]
            )


    def row_map(i):
        return (lax.axis_index("y") * NCHUNK + i, 0)

    return pl.pallas_call(
        body,
        grid=(NCHUNK,),
        out_shape=jax.ShapeDtypeStruct((2, d), jnp.float32),
        in_specs=[
            pl.BlockSpec((chunk, d), row_map),
            pl.BlockSpec((chunk, d), row_map),
        ],
        out_specs=pl.BlockSpec((2, d), lambda i: (0, 0)),
        scratch_shapes=[
            pltpu.VMEM((2, d), jnp.float32),
            pltpu.VMEM((N_DEV, 2, d), jnp.float32),
            pltpu.SemaphoreType.DMA((N_DEV,)),
            pltpu.SemaphoreType.DMA((N_DEV,)),
        ],
        compiler_params=pltpu.CompilerParams(collective_id=0),
    )(x, dy)


# device time: 16620 ns/iter; 1.2232x vs baseline; 1.1285x over previous
import jax
import jax.numpy as jnp
from jax import lax
from jax.experimental import pallas as pl
from jax.experimental.pallas import tpu as pltpu

N_DEV = 4
EPS = 1e-5
NCHUNK = 2


def kernel(x, dy, gamma):
    del gamma
    m, d = x.shape
    half = m // 2
    chunk = half // NCHUNK

    def body(x_ref, dy_ref, out_ref, acc_ref, comm_ref, send_sems, recv_sems):
        i = pl.program_id(0)
        my_x = lax.axis_index("x")
        my_y = lax.axis_index("y")
        my_pos = my_x * 2 + my_y
        peers = [(1 - my_x, my_y), (my_x, 1 - my_y), (1 - my_x, 1 - my_y)]

        barrier = pltpu.get_barrier_semaphore()

        @pl.when(i == 0)
        def _():
            for (px, py) in peers:
                pl.semaphore_signal(
                    barrier, inc=1,
                    device_id=(px, py), device_id_type=pl.DeviceIdType.MESH,
                )
            acc_ref[:, :] = jnp.zeros((2, d), jnp.float32)

        xb = x_ref[:, :]
        dyb = dy_ref[:, :]
        mu = jnp.mean(xb, axis=-1, keepdims=True)
        var = jnp.mean(jnp.square(xb - mu), axis=-1, keepdims=True)
        xhat = (xb - mu) * lax.rsqrt(var + EPS)
        dgamma = jnp.sum(dyb * xhat, axis=0, keepdims=True)
        dbeta = jnp.sum(dyb, axis=0, keepdims=True)
        acc_ref[:, :] += jnp.concatenate([dgamma, dbeta], axis=0)

        @pl.when(i == NCHUNK - 1)
        def _():
            comm_ref[my_pos] = acc_ref[:, :]
            pl.semaphore_wait(barrier, 3)

            sends = []
            for (px, py) in peers:
                q = px * 2 + py
                rdma = pltpu.make_async_remote_copy(
                    src_ref=comm_ref.at[my_pos],
                    dst_ref=comm_ref.at[my_pos],
                    send_sem=send_sems.at[q],
                    recv_sem=recv_sems.at[my_pos],
                    device_id=(px, py),
                    device_id_type=pl.DeviceIdType.MESH,
                )
                rdma.start()
                sends.append(rdma)

            for (px, py) in peers:
                q = px * 2 + py
                recv = pltpu.make_async_remote_copy(
                    src_ref=comm_ref.at[q],
                    dst_ref=comm_ref.at[q],
                    send_sem=send_sems.at[q],
                    recv_sem=recv_sems.at[q],
                    device_id=(px, py),
                    device_id_type=pl.DeviceIdType.MESH,
                )
                recv.wait_recv()

            out_ref[:, :] = (
                comm_ref[0] + comm_ref[1] + comm_ref[2] + comm_ref[3]
            )

            for rdma in sends:
                rdma.wait_send()

    def row_map(i):
        return (lax.axis_index("y") * NCHUNK + i, 0)

    return pl.pallas_call(
        body,
        grid=(NCHUNK,),
        out_shape=jax.ShapeDtypeStruct((2, d), jnp.float32),
        in_specs=[
            pl.BlockSpec((chunk, d), row_map),
            pl.BlockSpec((chunk, d), row_map),
        ],
        out_specs=pl.BlockSpec((2, d), lambda i: (0, 0)),
        scratch_shapes=[
            pltpu.VMEM((2, d), jnp.float32),
            pltpu.VMEM((N_DEV, 2, d), jnp.float32),
            pltpu.SemaphoreType.DMA((N_DEV,)),
            pltpu.SemaphoreType.DMA((N_DEV,)),
        ],
        compiler_params=pltpu.CompilerParams(collective_id=0),
    )(x, dy)
